# Initial kernel scaffold; baseline (speedup 1.0000x reference)
import jax
import jax.numpy as jnp
from jax.experimental import pallas as pl


def kernel(x, edge_index, W1, b1, W2, b2, W3, b3):
    raise NotImplementedError("write your pallas kernel here")


# speedup vs baseline: 12.7278x; 12.7278x over previous
import functools

import jax
import jax.numpy as jnp
from jax import lax
from jax.experimental import pallas as pl
from jax.experimental.pallas import tpu as pltpu
from jax.experimental.pallas import tpu_sc as plsc

N_NODES = 10000
N_EDGES = 320000
CH = 128
NC = 2
NS = 16
NW = NC * NS
EPW = N_EDGES // NW
WIN = 80
NWIN = EPW // WIN
NP = 10240
RPT = NP // NS

_mesh = plsc.VectorSubcoreMesh(core_axis_name="c", subcore_axis_name="s")


def _deg_body(dst_hbm, out_hbm, idx_v, ones_v, zero_v, deg_sh, sem):
    c = lax.axis_index("c")
    s = lax.axis_index("s")
    wid = c * NS + s

    @pl.when(s == 0)
    def _():
        def zb(i, _):
            zero_v[pl.ds(i * 16, 16)] = jnp.zeros((16,), jnp.float32)
            return 0
        lax.fori_loop(0, N_NODES // 16, zb, 0)
        pltpu.sync_copy(zero_v, deg_sh)

    for j in range(WIN // 16):
        ones_v[pl.ds(j * 16, 16)] = jnp.full((16,), 1.0, jnp.float32)
    plsc.subcore_barrier()

    def body(w, _):
        base = wid * EPW + w * WIN
        pltpu.sync_copy(dst_hbm.at[pl.ds(base, WIN)], idx_v)
        pltpu.sync_copy(ones_v, deg_sh.at[idx_v], add=True)
        return 0
    lax.fori_loop(0, NWIN, body, 0)

    plsc.subcore_barrier()

    @pl.when(s == 0)
    def _():
        pltpu.sync_copy(deg_sh, zero_v)
        pltpu.sync_copy(zero_v, out_hbm.at[pl.ds(c * N_NODES, N_NODES)])


_deg_kernel = functools.partial(
    pl.kernel,
    mesh=_mesh,
    out_type=jax.ShapeDtypeStruct((NC * N_NODES,), jnp.float32),
    scratch_types=[
        pltpu.VMEM((WIN,), jnp.int32),
        pltpu.VMEM((WIN,), jnp.float32),
        pltpu.VMEM((N_NODES,), jnp.float32),
        pltpu.VMEM_SHARED((N_NODES,), jnp.float32),
        pltpu.SemaphoreType.DMA,
    ],
)(_deg_body)


def _agg_body(hp_hbm, src_hbm, dst_hbm, out_hbm,
              isrc, idst, rows, zrow, wout, acc_sh, sem):
    c = lax.axis_index("c")
    s = lax.axis_index("s")
    wid = c * NS + s

    for r in range(zrow.shape[0]):
        for j in range(CH // 16):
            zrow[r, pl.ds(j * 16, 16)] = jnp.zeros((16,), jnp.float32)

    zr = zrow.shape[0]
    def zb(k, _):
        pltpu.sync_copy(zrow, acc_sh.at[pl.ds(s * RPT + k * zr, zr)])
        return 0
    lax.fori_loop(0, RPT // zr, zb, 0)
    plsc.subcore_barrier()

    def body(w, _):
        base = wid * EPW + w * WIN
        pltpu.sync_copy(src_hbm.at[pl.ds(base, WIN)], isrc)
        pltpu.sync_copy(dst_hbm.at[pl.ds(base, WIN)], idst)
        pltpu.async_copy(hp_hbm.at[isrc], rows, sem).wait()
        pltpu.sync_copy(rows, acc_sh.at[idst], add=True)
        return 0
    lax.fori_loop(0, NWIN, body, 0)

    plsc.subcore_barrier()

    def wb(k, _):
        pltpu.sync_copy(acc_sh.at[pl.ds(s * RPT + k * 128, 128)], wout)
        pltpu.sync_copy(wout, out_hbm.at[pl.ds(c * NP + s * RPT + k * 128, 128)])
        return 0
    lax.fori_loop(0, RPT // 128, wb, 0)


_agg_kernel = functools.partial(
    pl.kernel,
    mesh=_mesh,
    out_type=jax.ShapeDtypeStruct((NC * NP, CH), jnp.float32),
    scratch_types=[
        pltpu.VMEM((WIN,), jnp.int32),
        pltpu.VMEM((WIN,), jnp.int32),
        pltpu.VMEM((WIN, CH), jnp.float32),
        pltpu.VMEM((32, CH), jnp.float32),
        pltpu.VMEM((128, CH), jnp.float32),
        pltpu.VMEM_SHARED((NP, CH), jnp.float32),
        pltpu.SemaphoreType.DMA,
    ],
)(_agg_body)


_R = 1000


def _mm_scale_body(x_ref, w_ref, d0_ref, d1_ref, o_ref):
    dis = lax.rsqrt(d0_ref[...] + d1_ref[...] + 1.0)
    o_ref[...] = jnp.dot(x_ref[...], w_ref[...],
                         preferred_element_type=jnp.float32) * dis


def _mm_scale(x, w, d0, d1):
    return pl.pallas_call(
        _mm_scale_body,
        grid=(N_NODES // _R,),
        in_specs=[
            pl.BlockSpec((_R, CH), lambda i: (i, 0)),
            pl.BlockSpec((CH, CH), lambda i: (0, 0)),
            pl.BlockSpec((_R, 1), lambda i: (i, 0)),
            pl.BlockSpec((_R, 1), lambda i: (i, 0)),
        ],
        out_specs=pl.BlockSpec((_R, CH), lambda i: (i, 0)),
        out_shape=jax.ShapeDtypeStruct((N_NODES, CH), jnp.float32),
    )(x, w, d0, d1)


def _comb_mm_body(a0_ref, a1_ref, hp_ref, d0_ref, d1_ref, b_ref, w_ref, o_ref):
    dis = lax.rsqrt(d0_ref[...] + d1_ref[...] + 1.0)
    t = (a0_ref[...] + a1_ref[...] + hp_ref[...]) * dis + b_ref[...]
    t = jnp.maximum(t, 0.0)
    o_ref[...] = jnp.dot(t, w_ref[...],
                         preferred_element_type=jnp.float32) * dis


def _comb_mm(a0, a1, hp, d0, d1, b, w):
    return pl.pallas_call(
        _comb_mm_body,
        grid=(N_NODES // _R,),
        in_specs=[
            pl.BlockSpec((_R, CH), lambda i: (i, 0)),
            pl.BlockSpec((_R, CH), lambda i: (i, 0)),
            pl.BlockSpec((_R, CH), lambda i: (i, 0)),
            pl.BlockSpec((_R, 1), lambda i: (i, 0)),
            pl.BlockSpec((_R, 1), lambda i: (i, 0)),
            pl.BlockSpec((1, CH), lambda i: (0, 0)),
            pl.BlockSpec((CH, CH), lambda i: (0, 0)),
        ],
        out_specs=pl.BlockSpec((_R, CH), lambda i: (i, 0)),
        out_shape=jax.ShapeDtypeStruct((N_NODES, CH), jnp.float32),
    )(a0, a1, hp, d0, d1, b, w)


_OUT_PAD = 8


def _final_body(a0_ref, a1_ref, hp_ref, d0_ref, d1_ref, b_ref, w_ref,
                b3_ref, o_ref):
    dis = lax.rsqrt(d0_ref[...] + d1_ref[...] + 1.0)
    t = (a0_ref[...] + a1_ref[...] + hp_ref[...]) * dis + b_ref[...]
    t = jnp.maximum(t, 0.0)
    o_ref[...] = jnp.dot(t, w_ref[...],
                         preferred_element_type=jnp.float32) + b3_ref[...]


def _final(a0, a1, hp, d0, d1, b, w3p, b3p):
    return pl.pallas_call(
        _final_body,
        grid=(N_NODES // _R,),
        in_specs=[
            pl.BlockSpec((_R, CH), lambda i: (i, 0)),
            pl.BlockSpec((_R, CH), lambda i: (i, 0)),
            pl.BlockSpec((_R, CH), lambda i: (i, 0)),
            pl.BlockSpec((_R, 1), lambda i: (i, 0)),
            pl.BlockSpec((_R, 1), lambda i: (i, 0)),
            pl.BlockSpec((1, CH), lambda i: (0, 0)),
            pl.BlockSpec((CH, _OUT_PAD), lambda i: (0, 0)),
            pl.BlockSpec((1, _OUT_PAD), lambda i: (0, 0)),
        ],
        out_specs=pl.BlockSpec((_R, _OUT_PAD), lambda i: (i, 0)),
        out_shape=jax.ShapeDtypeStruct((N_NODES, _OUT_PAD), jnp.float32),
    )(a0, a1, hp, d0, d1, b, w3p, b3p)


def kernel(x, edge_index, W1, b1, W2, b2, W3, b3):
    src = edge_index[0].astype(jnp.int32)
    dst = edge_index[1].astype(jnp.int32)

    deg = _deg_kernel(dst)
    d0 = deg[:N_NODES].reshape(N_NODES, 1)
    d1 = deg[N_NODES:].reshape(N_NODES, 1)

    b1r = b1.reshape(1, CH)
    b2r = b2.reshape(1, CH)
    w3p = jnp.zeros((CH, _OUT_PAD), jnp.float32).at[:, :3].set(W3)
    b3p = jnp.zeros((1, _OUT_PAD), jnp.float32).at[0, :3].set(b3)

    h1p = _mm_scale(x, W1, d0, d1)
    agg1 = _agg_kernel(h1p, src, dst)
    h2p = _comb_mm(agg1[:N_NODES], agg1[NP:NP + N_NODES], h1p, d0, d1, b1r, W2)
    agg2 = _agg_kernel(h2p, src, dst)
    outp = _final(agg2[:N_NODES], agg2[NP:NP + N_NODES], h2p, d0, d1, b2r,
                  w3p, b3p)
    return outp[:, :3]


# speedup vs baseline: 27.2194x; 2.1386x over previous
import functools

import jax
import jax.numpy as jnp
from jax import lax
from jax.experimental import pallas as pl
from jax.experimental.pallas import tpu as pltpu
from jax.experimental.pallas import tpu_sc as plsc

N_NODES = 10000
N_EDGES = 320000
CH = 128
NC = 2
NS = 16
NW = NC * NS
EPW = N_EDGES // NW
WIN = 80
NWIN = EPW // WIN
NP = 10112
RPT = NP // NS

_mesh = plsc.VectorSubcoreMesh(core_axis_name="c", subcore_axis_name="s")


def _deg_body(dst3_hbm, out_hbm, idx2, ones_v, zero_v, deg_sh, sem):
    c = lax.axis_index("c")
    s = lax.axis_index("s")
    wid = c * NS + s

    @pl.when(s == 0)
    def _():
        def zb(i, _):
            zero_v[pl.ds(i * 16, 16)] = jnp.zeros((16,), jnp.float32)
            return 0
        lax.fori_loop(0, N_NODES // 16, zb, 0)
        pltpu.sync_copy(zero_v, deg_sh)

    for j in range(WIN // 16):
        ones_v[pl.ds(j * 16, 16)] = jnp.full((16,), 1.0, jnp.float32)
    pltpu.sync_copy(dst3_hbm.at[wid], idx2)
    plsc.subcore_barrier()

    def body(w, _):
        pltpu.sync_copy(ones_v, deg_sh.at[idx2.at[w]], add=True)
        return 0
    lax.fori_loop(0, NWIN, body, 0)

    plsc.subcore_barrier()

    @pl.when(s == 0)
    def _():
        pltpu.sync_copy(deg_sh, zero_v)
        pltpu.sync_copy(zero_v, out_hbm.at[pl.ds(c * N_NODES, N_NODES)])


_deg_kernel = functools.partial(
    pl.kernel,
    mesh=_mesh,
    out_type=jax.ShapeDtypeStruct((NC * N_NODES,), jnp.float32),
    scratch_types=[
        pltpu.VMEM((NWIN, WIN), jnp.int32),
        pltpu.VMEM((WIN,), jnp.float32),
        pltpu.VMEM((N_NODES,), jnp.float32),
        pltpu.VMEM_SHARED((N_NODES,), jnp.float32),
        pltpu.SemaphoreType.DMA,
    ],
)(_deg_body)


def _agg_body(hp_hbm, src2_hbm, dst3_hbm, out_hbm,
              isrc2, idst2, rows0, rows1, wout, acc_sh, sg0, sg1):
    c = lax.axis_index("c")
    s = lax.axis_index("s")
    wid = c * NS + s

    for r in range(wout.shape[0]):
        for j in range(CH // 16):
            wout[r, pl.ds(j * 16, 16)] = jnp.zeros((16,), jnp.float32)

    def zb(k, _):
        pltpu.sync_copy(wout, acc_sh.at[pl.ds(s * RPT + k * 16, 16)])
        return 0
    lax.fori_loop(0, RPT // 16, zb, 0)
    pltpu.sync_copy(wout.at[pl.ds(0, 8)],
                    acc_sh.at[pl.ds(s * RPT + (RPT // 16) * 16, 8)])

    pltpu.sync_copy(src2_hbm.at[wid], isrc2)
    pltpu.sync_copy(dst3_hbm.at[wid], idst2)
    plsc.subcore_barrier()

    def gather(w, buf, sem):
        pltpu.async_copy(hp_hbm.at[isrc2.at[pl.ds(w * WIN, WIN)]], buf, sem)

    def gwait(buf, sem):
        pltpu.make_async_copy(hp_hbm.at[isrc2.at[pl.ds(0, WIN)]], buf,
                              sem).wait()

    def scat(w, buf):
        pltpu.sync_copy(buf, acc_sh.at[idst2.at[w]], add=True)

    gather(0, rows0, sg0)
    gather(1, rows1, sg1)

    def body(k, _):
        w = 2 * k
        gwait(rows0, sg0)
        scat(w, rows0)
        gather(w + 2, rows0, sg0)
        gwait(rows1, sg1)
        scat(w + 1, rows1)
        gather(w + 3, rows1, sg1)
        return 0
    lax.fori_loop(0, (NWIN - 3) // 2, body, 0)

    gwait(rows0, sg0)
    scat(NWIN - 3, rows0)
    gather(NWIN - 1, rows0, sg0)
    gwait(rows1, sg1)
    scat(NWIN - 2, rows1)
    gwait(rows0, sg0)
    scat(NWIN - 1, rows0)

    plsc.subcore_barrier()

    def wb(k, _):
        pltpu.sync_copy(acc_sh.at[pl.ds(s * RPT + k * 16, 16)], wout)
        pltpu.sync_copy(wout, out_hbm.at[pl.ds(c * NP + s * RPT + k * 16, 16)])
        return 0
    lax.fori_loop(0, RPT // 16, wb, 0)
    tail = s * RPT + (RPT // 16) * 16
    pltpu.sync_copy(acc_sh.at[pl.ds(tail, 8)], wout.at[pl.ds(0, 8)])
    pltpu.sync_copy(wout.at[pl.ds(0, 8)], out_hbm.at[pl.ds(c * NP + tail, 8)])


_agg_kernel = functools.partial(
    pl.kernel,
    mesh=_mesh,
    out_type=jax.ShapeDtypeStruct((NC * NP, CH), jnp.float32),
    scratch_types=[
        pltpu.VMEM((EPW,), jnp.int32),
        pltpu.VMEM((NWIN, WIN), jnp.int32),
        pltpu.VMEM((WIN, CH), jnp.float32),
        pltpu.VMEM((WIN, CH), jnp.float32),
        pltpu.VMEM((16, CH), jnp.float32),
        pltpu.VMEM_SHARED((NP, CH), jnp.float32),
        pltpu.SemaphoreType.DMA,
        pltpu.SemaphoreType.DMA,
    ],
)(_agg_body)


_R = 1000


def _mm_scale_body(x_ref, w_ref, d0_ref, d1_ref, o_ref):
    dis = lax.rsqrt(d0_ref[...] + d1_ref[...] + 1.0)
    o_ref[...] = jnp.dot(x_ref[...], w_ref[...],
                         preferred_element_type=jnp.float32) * dis


def _mm_scale(x, w, d0, d1):
    return pl.pallas_call(
        _mm_scale_body,
        grid=(N_NODES // _R,),
        in_specs=[
            pl.BlockSpec((_R, CH), lambda i: (i, 0)),
            pl.BlockSpec((CH, CH), lambda i: (0, 0)),
            pl.BlockSpec((_R, 1), lambda i: (i, 0)),
            pl.BlockSpec((_R, 1), lambda i: (i, 0)),
        ],
        out_specs=pl.BlockSpec((_R, CH), lambda i: (i, 0)),
        out_shape=jax.ShapeDtypeStruct((N_NODES, CH), jnp.float32),
    )(x, w, d0, d1)


def _comb_mm_body(a0_ref, a1_ref, hp_ref, d0_ref, d1_ref, b_ref, w_ref, o_ref):
    dis = lax.rsqrt(d0_ref[...] + d1_ref[...] + 1.0)
    t = (a0_ref[...] + a1_ref[...] + hp_ref[...]) * dis + b_ref[...]
    t = jnp.maximum(t, 0.0)
    o_ref[...] = jnp.dot(t, w_ref[...],
                         preferred_element_type=jnp.float32) * dis


def _comb_mm(a0, a1, hp, d0, d1, b, w):
    return pl.pallas_call(
        _comb_mm_body,
        grid=(N_NODES // _R,),
        in_specs=[
            pl.BlockSpec((_R, CH), lambda i: (i, 0)),
            pl.BlockSpec((_R, CH), lambda i: (i, 0)),
            pl.BlockSpec((_R, CH), lambda i: (i, 0)),
            pl.BlockSpec((_R, 1), lambda i: (i, 0)),
            pl.BlockSpec((_R, 1), lambda i: (i, 0)),
            pl.BlockSpec((1, CH), lambda i: (0, 0)),
            pl.BlockSpec((CH, CH), lambda i: (0, 0)),
        ],
        out_specs=pl.BlockSpec((_R, CH), lambda i: (i, 0)),
        out_shape=jax.ShapeDtypeStruct((N_NODES, CH), jnp.float32),
    )(a0, a1, hp, d0, d1, b, w)


_OUT_PAD = 8


def _final_body(a0_ref, a1_ref, hp_ref, d0_ref, d1_ref, b_ref, w_ref,
                b3_ref, o_ref):
    dis = lax.rsqrt(d0_ref[...] + d1_ref[...] + 1.0)
    t = (a0_ref[...] + a1_ref[...] + hp_ref[...]) * dis + b_ref[...]
    t = jnp.maximum(t, 0.0)
    o_ref[...] = jnp.dot(t, w_ref[...],
                         preferred_element_type=jnp.float32) + b3_ref[...]


def _final(a0, a1, hp, d0, d1, b, w3p, b3p):
    return pl.pallas_call(
        _final_body,
        grid=(N_NODES // _R,),
        in_specs=[
            pl.BlockSpec((_R, CH), lambda i: (i, 0)),
            pl.BlockSpec((_R, CH), lambda i: (i, 0)),
            pl.BlockSpec((_R, CH), lambda i: (i, 0)),
            pl.BlockSpec((_R, 1), lambda i: (i, 0)),
            pl.BlockSpec((_R, 1), lambda i: (i, 0)),
            pl.BlockSpec((1, CH), lambda i: (0, 0)),
            pl.BlockSpec((CH, _OUT_PAD), lambda i: (0, 0)),
            pl.BlockSpec((1, _OUT_PAD), lambda i: (0, 0)),
        ],
        out_specs=pl.BlockSpec((_R, _OUT_PAD), lambda i: (i, 0)),
        out_shape=jax.ShapeDtypeStruct((N_NODES, _OUT_PAD), jnp.float32),
    )(a0, a1, hp, d0, d1, b, w3p, b3p)


def kernel(x, edge_index, W1, b1, W2, b2, W3, b3):
    src = edge_index[0].astype(jnp.int32).reshape(NW, EPW)
    dst = edge_index[1].astype(jnp.int32).reshape(NW, NWIN, WIN)

    deg = _deg_kernel(dst)
    d0 = deg[:N_NODES].reshape(N_NODES, 1)
    d1 = deg[N_NODES:].reshape(N_NODES, 1)

    b1r = b1.reshape(1, CH)
    b2r = b2.reshape(1, CH)
    w3p = jnp.zeros((CH, _OUT_PAD), jnp.float32).at[:, :3].set(W3)
    b3p = jnp.zeros((1, _OUT_PAD), jnp.float32).at[0, :3].set(b3)

    h1p = _mm_scale(x, W1, d0, d1)
    agg1 = _agg_kernel(h1p, src, dst)
    h2p = _comb_mm(agg1[:N_NODES], agg1[NP:NP + N_NODES], h1p, d0, d1,
                   b1r, W2)
    agg2 = _agg_kernel(h2p, src, dst)
    outp = _final(agg2[:N_NODES], agg2[NP:NP + N_NODES], h2p, d0, d1,
                  b2r, w3p, b3p)
    return outp[:, :3]


# speedup vs baseline: 29.0684x; 1.0679x over previous
import functools

import jax
import jax.numpy as jnp
from jax import lax
from jax.experimental import pallas as pl
from jax.experimental.pallas import tpu as pltpu
from jax.experimental.pallas import tpu_sc as plsc

N_NODES = 10000
N_EDGES = 320000
CH = 128
NC = 2
NS = 16
NW = NC * NS
EPW = N_EDGES // NW
WIN = 80
NWIN = EPW // WIN
NP = 10112
RPT = NP // NS

_mesh = plsc.VectorSubcoreMesh(core_axis_name="c", subcore_axis_name="s")


def _deg_body(dst3_hbm, out_hbm, idx2, ones_v, zero_v, deg_sh, sem):
    c = lax.axis_index("c")
    s = lax.axis_index("s")
    wid = c * NS + s

    @pl.when(s == 0)
    def _():
        def zb(i, _):
            zero_v[pl.ds(i * 16, 16)] = jnp.zeros((16,), jnp.float32)
            return 0
        lax.fori_loop(0, N_NODES // 16, zb, 0)
        pltpu.sync_copy(zero_v, deg_sh)

    for j in range(WIN // 16):
        ones_v[pl.ds(j * 16, 16)] = jnp.full((16,), 1.0, jnp.float32)
    pltpu.sync_copy(dst3_hbm.at[wid], idx2)
    plsc.subcore_barrier()

    def body(w, _):
        pltpu.sync_copy(ones_v, deg_sh.at[idx2.at[w]], add=True)
        return 0
    lax.fori_loop(0, NWIN, body, 0)

    plsc.subcore_barrier()

    @pl.when(s == 0)
    def _():
        pltpu.sync_copy(deg_sh, zero_v)
        pltpu.sync_copy(zero_v, out_hbm.at[pl.ds(c * N_NODES, N_NODES)])


_deg_kernel = functools.partial(
    pl.kernel,
    mesh=_mesh,
    out_type=jax.ShapeDtypeStruct((NC * N_NODES,), jnp.float32),
    scratch_types=[
        pltpu.VMEM((NWIN, WIN), jnp.int32),
        pltpu.VMEM((WIN,), jnp.float32),
        pltpu.VMEM((N_NODES,), jnp.float32),
        pltpu.VMEM_SHARED((N_NODES,), jnp.float32),
        pltpu.SemaphoreType.DMA,
    ],
)(_deg_body)


def _agg_body(hp_hbm, src2_hbm, dst3_hbm, out_hbm,
              isrc2, idst2, rows0, rows1, wout, acc_sh, sg0, sg1, sz):
    c = lax.axis_index("c")
    s = lax.axis_index("s")
    wid = c * NS + s

    pltpu.async_copy(src2_hbm.at[wid], isrc2, sg0)
    pltpu.async_copy(dst3_hbm.at[wid], idst2, sg1)

    for r in range(wout.shape[0]):
        for j in range(CH // 16):
            wout[r, pl.ds(j * 16, 16)] = jnp.zeros((16,), jnp.float32)

    def zb(k, _):
        pltpu.async_copy(wout, acc_sh.at[pl.ds(s * RPT + k * 16, 16)], sz)
        return 0
    lax.fori_loop(0, RPT // 16, zb, 0)
    pltpu.async_copy(wout.at[pl.ds(0, 8)],
                     acc_sh.at[pl.ds(s * RPT + (RPT // 16) * 16, 8)], sz)

    def zdrain(k, _):
        pltpu.make_async_copy(wout, acc_sh.at[pl.ds(s * RPT, 16)], sz).wait()
        return 0
    lax.fori_loop(0, RPT // 16, zdrain, 0)
    pltpu.make_async_copy(wout.at[pl.ds(0, 8)],
                          acc_sh.at[pl.ds(s * RPT, 8)], sz).wait()
    pltpu.make_async_copy(src2_hbm.at[wid], isrc2, sg0).wait()
    pltpu.make_async_copy(dst3_hbm.at[wid], idst2, sg1).wait()
    plsc.subcore_barrier()

    def gather(w, buf, sem):
        pltpu.async_copy(hp_hbm.at[isrc2.at[pl.ds(w * WIN, WIN)]], buf, sem)

    def gwait(buf, sem):
        pltpu.make_async_copy(hp_hbm.at[isrc2.at[pl.ds(0, WIN)]], buf,
                              sem).wait()

    def scat(w, buf):
        pltpu.sync_copy(buf, acc_sh.at[idst2.at[w]], add=True)

    gather(0, rows0, sg0)
    gather(1, rows1, sg1)

    def body(k, _):
        w = 2 * k
        gwait(rows0, sg0)
        scat(w, rows0)
        gather(w + 2, rows0, sg0)
        gwait(rows1, sg1)
        scat(w + 1, rows1)
        gather(w + 3, rows1, sg1)
        return 0
    lax.fori_loop(0, (NWIN - 3) // 2, body, 0)

    gwait(rows0, sg0)
    scat(NWIN - 3, rows0)
    gather(NWIN - 1, rows0, sg0)
    gwait(rows1, sg1)
    scat(NWIN - 2, rows1)
    gwait(rows0, sg0)
    scat(NWIN - 1, rows0)

    plsc.subcore_barrier()

    bufs = (rows0, rows1)
    sems = (sg0, sg1)
    sizes = [80] * 7 + [72]
    offs = [80 * k for k in range(8)]
    pltpu.sync_copy(acc_sh.at[pl.ds(s * RPT, 80)], rows0)
    for k in range(8):
        b, sm = bufs[k % 2], sems[k % 2]
        n = sizes[k]
        pltpu.async_copy(b.at[pl.ds(0, n)],
                         out_hbm.at[pl.ds(c * NP + s * RPT + offs[k], n)], sm)
        if k + 1 < 8:
            nb = bufs[(k + 1) % 2]
            pltpu.sync_copy(
                acc_sh.at[pl.ds(s * RPT + offs[k + 1], sizes[k + 1])],
                nb.at[pl.ds(0, sizes[k + 1])])
        pltpu.make_async_copy(
            b.at[pl.ds(0, n)],
            out_hbm.at[pl.ds(c * NP + s * RPT + offs[k], n)], sm).wait()


_agg_kernel = functools.partial(
    pl.kernel,
    mesh=_mesh,
    out_type=jax.ShapeDtypeStruct((NC * NP, CH), jnp.float32),
    scratch_types=[
        pltpu.VMEM((EPW,), jnp.int32),
        pltpu.VMEM((NWIN, WIN), jnp.int32),
        pltpu.VMEM((WIN, CH), jnp.float32),
        pltpu.VMEM((WIN, CH), jnp.float32),
        pltpu.VMEM((16, CH), jnp.float32),
        pltpu.VMEM_SHARED((NP, CH), jnp.float32),
        pltpu.SemaphoreType.DMA,
        pltpu.SemaphoreType.DMA,
        pltpu.SemaphoreType.DMA,
    ],
)(_agg_body)


_R = 1000


def _mm_scale_body(x_ref, w_ref, d0_ref, d1_ref, o_ref):
    dis = lax.rsqrt(d0_ref[...] + d1_ref[...] + 1.0)
    o_ref[...] = jnp.dot(x_ref[...], w_ref[...],
                         preferred_element_type=jnp.float32) * dis


def _mm_scale(x, w, d0, d1):
    return pl.pallas_call(
        _mm_scale_body,
        grid=(N_NODES // _R,),
        in_specs=[
            pl.BlockSpec((_R, CH), lambda i: (i, 0)),
            pl.BlockSpec((CH, CH), lambda i: (0, 0)),
            pl.BlockSpec((_R, 1), lambda i: (i, 0)),
            pl.BlockSpec((_R, 1), lambda i: (i, 0)),
        ],
        out_specs=pl.BlockSpec((_R, CH), lambda i: (i, 0)),
        out_shape=jax.ShapeDtypeStruct((N_NODES, CH), jnp.float32),
    )(x, w, d0, d1)


def _comb_mm_body(a0_ref, a1_ref, hp_ref, d0_ref, d1_ref, b_ref, w_ref, o_ref):
    dis = lax.rsqrt(d0_ref[...] + d1_ref[...] + 1.0)
    t = (a0_ref[...] + a1_ref[...] + hp_ref[...]) * dis + b_ref[...]
    t = jnp.maximum(t, 0.0)
    o_ref[...] = jnp.dot(t, w_ref[...],
                         preferred_element_type=jnp.float32) * dis


def _comb_mm(a0, a1, hp, d0, d1, b, w):
    return pl.pallas_call(
        _comb_mm_body,
        grid=(N_NODES // _R,),
        in_specs=[
            pl.BlockSpec((_R, CH), lambda i: (i, 0)),
            pl.BlockSpec((_R, CH), lambda i: (i, 0)),
            pl.BlockSpec((_R, CH), lambda i: (i, 0)),
            pl.BlockSpec((_R, 1), lambda i: (i, 0)),
            pl.BlockSpec((_R, 1), lambda i: (i, 0)),
            pl.BlockSpec((1, CH), lambda i: (0, 0)),
            pl.BlockSpec((CH, CH), lambda i: (0, 0)),
        ],
        out_specs=pl.BlockSpec((_R, CH), lambda i: (i, 0)),
        out_shape=jax.ShapeDtypeStruct((N_NODES, CH), jnp.float32),
    )(a0, a1, hp, d0, d1, b, w)


_OUT_PAD = 8


def _final_body(a0_ref, a1_ref, hp_ref, d0_ref, d1_ref, b_ref, w_ref,
                b3_ref, o_ref):
    dis = lax.rsqrt(d0_ref[...] + d1_ref[...] + 1.0)
    t = (a0_ref[...] + a1_ref[...] + hp_ref[...]) * dis + b_ref[...]
    t = jnp.maximum(t, 0.0)
    o_ref[...] = jnp.dot(t, w_ref[...],
                         preferred_element_type=jnp.float32) + b3_ref[...]


def _final(a0, a1, hp, d0, d1, b, w3p, b3p):
    return pl.pallas_call(
        _final_body,
        grid=(N_NODES // _R,),
        in_specs=[
            pl.BlockSpec((_R, CH), lambda i: (i, 0)),
            pl.BlockSpec((_R, CH), lambda i: (i, 0)),
            pl.BlockSpec((_R, CH), lambda i: (i, 0)),
            pl.BlockSpec((_R, 1), lambda i: (i, 0)),
            pl.BlockSpec((_R, 1), lambda i: (i, 0)),
            pl.BlockSpec((1, CH), lambda i: (0, 0)),
            pl.BlockSpec((CH, _OUT_PAD), lambda i: (0, 0)),
            pl.BlockSpec((1, _OUT_PAD), lambda i: (0, 0)),
        ],
        out_specs=pl.BlockSpec((_R, _OUT_PAD), lambda i: (i, 0)),
        out_shape=jax.ShapeDtypeStruct((N_NODES, _OUT_PAD), jnp.float32),
    )(a0, a1, hp, d0, d1, b, w3p, b3p)


def kernel(x, edge_index, W1, b1, W2, b2, W3, b3):
    src = edge_index[0].astype(jnp.int32).reshape(NW, EPW)
    dst = edge_index[1].astype(jnp.int32).reshape(NW, NWIN, WIN)

    deg = _deg_kernel(dst)
    d0 = deg[:N_NODES].reshape(N_NODES, 1)
    d1 = deg[N_NODES:].reshape(N_NODES, 1)

    b1r = b1.reshape(1, CH)
    b2r = b2.reshape(1, CH)
    w3p = jnp.zeros((CH, _OUT_PAD), jnp.float32).at[:, :3].set(W3)
    b3p = jnp.zeros((1, _OUT_PAD), jnp.float32).at[0, :3].set(b3)

    h1p = _mm_scale(x, W1, d0, d1)
    agg1 = _agg_kernel(h1p, src, dst)
    h2p = _comb_mm(agg1[:N_NODES], agg1[NP:NP + N_NODES], h1p, d0, d1,
                   b1r, W2)
    agg2 = _agg_kernel(h2p, src, dst)
    outp = _final(agg2[:N_NODES], agg2[NP:NP + N_NODES], h2p, d0, d1,
                  b2r, w3p, b3p)
    return outp[:, :3]


# speedup vs baseline: 29.1824x; 1.0039x over previous
import functools

import jax
import jax.numpy as jnp
from jax import lax
from jax.experimental import pallas as pl
from jax.experimental.pallas import tpu as pltpu
from jax.experimental.pallas import tpu_sc as plsc

N_NODES = 10000
N_EDGES = 320000
CH = 128
NC = 2
NS = 16
NW = NC * NS
EPW = N_EDGES // NW
WIN = 80
NWIN = EPW // WIN
NP = 10112
RPT = NP // NS

_mesh = plsc.VectorSubcoreMesh(core_axis_name="c", subcore_axis_name="s")


def _deg_body(dst3_hbm, out_hbm, idx2, ones_v, zero_v, deg_sh, sem):
    c = lax.axis_index("c")
    s = lax.axis_index("s")
    wid = c * NS + s

    @pl.when(s == 0)
    def _():
        def zb(i, _):
            zero_v[pl.ds(i * 16, 16)] = jnp.zeros((16,), jnp.float32)
            return 0
        lax.fori_loop(0, N_NODES // 16, zb, 0)
        pltpu.sync_copy(zero_v, deg_sh)

    for j in range(WIN // 16):
        ones_v[pl.ds(j * 16, 16)] = jnp.full((16,), 1.0, jnp.float32)
    pltpu.sync_copy(dst3_hbm.at[wid], idx2)
    plsc.subcore_barrier()

    def body(w, _):
        pltpu.sync_copy(ones_v, deg_sh.at[idx2.at[w]], add=True)
        return 0
    lax.fori_loop(0, NWIN, body, 0)

    plsc.subcore_barrier()

    @pl.when(s == 0)
    def _():
        pltpu.sync_copy(deg_sh, zero_v)
        pltpu.sync_copy(zero_v, out_hbm.at[pl.ds(c * N_NODES, N_NODES)])


_deg_kernel = functools.partial(
    pl.kernel,
    mesh=_mesh,
    out_type=jax.ShapeDtypeStruct((NC * N_NODES,), jnp.float32),
    scratch_types=[
        pltpu.VMEM((NWIN, WIN), jnp.int32),
        pltpu.VMEM((WIN,), jnp.float32),
        pltpu.VMEM((N_NODES,), jnp.float32),
        pltpu.VMEM_SHARED((N_NODES,), jnp.float32),
        pltpu.SemaphoreType.DMA,
    ],
)(_deg_body)


def _agg_body(hp_hbm, src2_hbm, dst3_hbm, out_hbm,
              isrc2, idst2, rows0, rows1, wout, acc_sh, sg0, sg1, sz):
    c = lax.axis_index("c")
    s = lax.axis_index("s")
    wid = c * NS + s

    pltpu.async_copy(src2_hbm.at[wid], isrc2, sg0)
    pltpu.async_copy(dst3_hbm.at[wid], idst2, sg1)

    for r in range(wout.shape[0]):
        for j in range(CH // 16):
            wout[r, pl.ds(j * 16, 16)] = jnp.zeros((16,), jnp.float32)

    def zb(k, _):
        pltpu.async_copy(wout, acc_sh.at[pl.ds(s * RPT + k * 16, 16)], sz)
        return 0
    lax.fori_loop(0, RPT // 16, zb, 0)
    pltpu.async_copy(wout.at[pl.ds(0, 8)],
                     acc_sh.at[pl.ds(s * RPT + (RPT // 16) * 16, 8)], sz)

    def zdrain(k, _):
        pltpu.make_async_copy(wout, acc_sh.at[pl.ds(s * RPT, 16)], sz).wait()
        return 0
    lax.fori_loop(0, RPT // 16, zdrain, 0)
    pltpu.make_async_copy(wout.at[pl.ds(0, 8)],
                          acc_sh.at[pl.ds(s * RPT, 8)], sz).wait()
    pltpu.make_async_copy(src2_hbm.at[wid], isrc2, sg0).wait()
    pltpu.make_async_copy(dst3_hbm.at[wid], idst2, sg1).wait()
    plsc.subcore_barrier()

    def gather(w, buf, sem):
        pltpu.async_copy(hp_hbm.at[isrc2.at[pl.ds(w * WIN, WIN)]], buf, sem)

    def gwait(buf, sem):
        pltpu.make_async_copy(hp_hbm.at[isrc2.at[pl.ds(0, WIN)]], buf,
                              sem).wait()

    def scat(w, buf):
        pltpu.sync_copy(buf, acc_sh.at[idst2.at[w]], add=True)

    gather(0, rows0, sg0)
    gather(1, rows1, sg1)

    def body(k, _):
        w = 2 * k
        gwait(rows0, sg0)
        scat(w, rows0)
        gather(w + 2, rows0, sg0)
        gwait(rows1, sg1)
        scat(w + 1, rows1)
        gather(w + 3, rows1, sg1)
        return 0
    lax.fori_loop(0, (NWIN - 3) // 2, body, 0)

    gwait(rows0, sg0)
    scat(NWIN - 3, rows0)
    gather(NWIN - 1, rows0, sg0)
    gwait(rows1, sg1)
    scat(NWIN - 2, rows1)
    gwait(rows0, sg0)
    scat(NWIN - 1, rows0)

    plsc.subcore_barrier()

    bufs = (rows0, rows1)
    sems = (sg0, sg1)
    sizes = [80] * 7 + [72]
    offs = [80 * k for k in range(8)]
    pltpu.sync_copy(acc_sh.at[pl.ds(s * RPT, 80)], rows0)
    for k in range(8):
        b, sm = bufs[k % 2], sems[k % 2]
        n = sizes[k]
        pltpu.async_copy(b.at[pl.ds(0, n)],
                         out_hbm.at[pl.ds(c * NP + s * RPT + offs[k], n)], sm)
        if k + 1 < 8:
            nb = bufs[(k + 1) % 2]
            pltpu.sync_copy(
                acc_sh.at[pl.ds(s * RPT + offs[k + 1], sizes[k + 1])],
                nb.at[pl.ds(0, sizes[k + 1])])
        pltpu.make_async_copy(
            b.at[pl.ds(0, n)],
            out_hbm.at[pl.ds(c * NP + s * RPT + offs[k], n)], sm).wait()


_agg_kernel = functools.partial(
    pl.kernel,
    mesh=_mesh,
    out_type=jax.ShapeDtypeStruct((NC * NP, CH), jnp.float32),
    scratch_types=[
        pltpu.VMEM((EPW,), jnp.int32),
        pltpu.VMEM((NWIN, WIN), jnp.int32),
        pltpu.VMEM((WIN, CH), jnp.float32),
        pltpu.VMEM((WIN, CH), jnp.float32),
        pltpu.VMEM((16, CH), jnp.float32),
        pltpu.VMEM_SHARED((NP, CH), jnp.float32),
        pltpu.SemaphoreType.DMA,
        pltpu.SemaphoreType.DMA,
        pltpu.SemaphoreType.DMA,
    ],
)(_agg_body)


_R = 1000


def _mm_body(x_ref, w_ref, o_ref):
    o_ref[...] = jnp.dot(x_ref[...], w_ref[...],
                         preferred_element_type=jnp.float32)


def _mm(x, w):
    return pl.pallas_call(
        _mm_body,
        grid=(N_NODES // _R,),
        in_specs=[
            pl.BlockSpec((_R, CH), lambda i: (i, 0)),
            pl.BlockSpec((CH, CH), lambda i: (0, 0)),
        ],
        out_specs=pl.BlockSpec((_R, CH), lambda i: (i, 0)),
        out_shape=jax.ShapeDtypeStruct((N_NODES, CH), jnp.float32),
    )(x, w)


def _scale_body(h_ref, d0_ref, d1_ref, o_ref):
    dis = lax.rsqrt(d0_ref[...] + d1_ref[...] + 1.0)
    o_ref[...] = h_ref[...] * dis


def _scale(h, d0, d1):
    return pl.pallas_call(
        _scale_body,
        grid=(N_NODES // _R,),
        in_specs=[
            pl.BlockSpec((_R, CH), lambda i: (i, 0)),
            pl.BlockSpec((_R, 1), lambda i: (i, 0)),
            pl.BlockSpec((_R, 1), lambda i: (i, 0)),
        ],
        out_specs=pl.BlockSpec((_R, CH), lambda i: (i, 0)),
        out_shape=jax.ShapeDtypeStruct((N_NODES, CH), jnp.float32),
    )(h, d0, d1)


def _comb_mm_body(a0_ref, a1_ref, hp_ref, d0_ref, d1_ref, b_ref, w_ref, o_ref):
    dis = lax.rsqrt(d0_ref[...] + d1_ref[...] + 1.0)
    t = (a0_ref[...] + a1_ref[...] + hp_ref[...]) * dis + b_ref[...]
    t = jnp.maximum(t, 0.0)
    o_ref[...] = jnp.dot(t, w_ref[...],
                         preferred_element_type=jnp.float32) * dis


def _comb_mm(a0, a1, hp, d0, d1, b, w):
    return pl.pallas_call(
        _comb_mm_body,
        grid=(N_NODES // _R,),
        in_specs=[
            pl.BlockSpec((_R, CH), lambda i: (i, 0)),
            pl.BlockSpec((_R, CH), lambda i: (i, 0)),
            pl.BlockSpec((_R, CH), lambda i: (i, 0)),
            pl.BlockSpec((_R, 1), lambda i: (i, 0)),
            pl.BlockSpec((_R, 1), lambda i: (i, 0)),
            pl.BlockSpec((1, CH), lambda i: (0, 0)),
            pl.BlockSpec((CH, CH), lambda i: (0, 0)),
        ],
        out_specs=pl.BlockSpec((_R, CH), lambda i: (i, 0)),
        out_shape=jax.ShapeDtypeStruct((N_NODES, CH), jnp.float32),
    )(a0, a1, hp, d0, d1, b, w)


_OUT_PAD = 8


def _final_body(a0_ref, a1_ref, hp_ref, d0_ref, d1_ref, b_ref, w_ref,
                b3_ref, o_ref):
    dis = lax.rsqrt(d0_ref[...] + d1_ref[...] + 1.0)
    t = (a0_ref[...] + a1_ref[...] + hp_ref[...]) * dis + b_ref[...]
    t = jnp.maximum(t, 0.0)
    o_ref[...] = jnp.dot(t, w_ref[...],
                         preferred_element_type=jnp.float32) + b3_ref[...]


def _final(a0, a1, hp, d0, d1, b, w3p, b3p):
    return pl.pallas_call(
        _final_body,
        grid=(N_NODES // _R,),
        in_specs=[
            pl.BlockSpec((_R, CH), lambda i: (i, 0)),
            pl.BlockSpec((_R, CH), lambda i: (i, 0)),
            pl.BlockSpec((_R, CH), lambda i: (i, 0)),
            pl.BlockSpec((_R, 1), lambda i: (i, 0)),
            pl.BlockSpec((_R, 1), lambda i: (i, 0)),
            pl.BlockSpec((1, CH), lambda i: (0, 0)),
            pl.BlockSpec((CH, _OUT_PAD), lambda i: (0, 0)),
            pl.BlockSpec((1, _OUT_PAD), lambda i: (0, 0)),
        ],
        out_specs=pl.BlockSpec((_R, _OUT_PAD), lambda i: (i, 0)),
        out_shape=jax.ShapeDtypeStruct((N_NODES, _OUT_PAD), jnp.float32),
    )(a0, a1, hp, d0, d1, b, w3p, b3p)


def kernel(x, edge_index, W1, b1, W2, b2, W3, b3):
    src = edge_index[0].astype(jnp.int32).reshape(NW, EPW)
    dst = edge_index[1].astype(jnp.int32).reshape(NW, NWIN, WIN)

    deg = _deg_kernel(dst)
    d0 = deg[:N_NODES].reshape(N_NODES, 1)
    d1 = deg[N_NODES:].reshape(N_NODES, 1)

    b1r = b1.reshape(1, CH)
    b2r = b2.reshape(1, CH)
    w3p = jnp.zeros((CH, _OUT_PAD), jnp.float32).at[:, :3].set(W3)
    b3p = jnp.zeros((1, _OUT_PAD), jnp.float32).at[0, :3].set(b3)

    h1p = _scale(_mm(x, W1), d0, d1)
    agg1 = _agg_kernel(h1p, src, dst)
    h2p = _comb_mm(agg1[:N_NODES], agg1[NP:NP + N_NODES], h1p, d0, d1,
                   b1r, W2)
    agg2 = _agg_kernel(h2p, src, dst)
    outp = _final(agg2[:N_NODES], agg2[NP:NP + N_NODES], h2p, d0, d1,
                  b2r, w3p, b3p)
    return outp[:, :3]


# speedup vs baseline: 30.3694x; 1.0407x over previous
import functools

import jax
import jax.numpy as jnp
from jax import lax
from jax.experimental import pallas as pl
from jax.experimental.pallas import tpu as pltpu
from jax.experimental.pallas import tpu_sc as plsc

N_NODES = 10000
N_EDGES = 320000
CH = 128
NC = 2
NS = 16
NW = NC * NS
EPW = N_EDGES // NW
WIN = 80
NWIN = EPW // WIN
NP = 10112
RPT = NP // NS

_mesh = plsc.VectorSubcoreMesh(core_axis_name="c", subcore_axis_name="s")


def _deg_body(ei3_hbm, out_hbm, idx2, ones_v, zero_v, deg_sh, sem):
    c = lax.axis_index("c")
    s = lax.axis_index("s")
    wid = c * NS + s

    @pl.when(s == 0)
    def _():
        def zb(i, _):
            zero_v[pl.ds(i * 16, 16)] = jnp.zeros((16,), jnp.float32)
            return 0
        lax.fori_loop(0, NP // 16, zb, 0)
        pltpu.sync_copy(zero_v.at[pl.ds(0, N_NODES)], deg_sh)

    for j in range(WIN // 16):
        ones_v[pl.ds(j * 16, 16)] = jnp.full((16,), 1.0, jnp.float32)
    pltpu.sync_copy(ei3_hbm.at[1, wid], idx2)
    plsc.subcore_barrier()

    def body(w, _):
        pltpu.sync_copy(ones_v, deg_sh.at[idx2.at[w]], add=True)
        return 0
    lax.fori_loop(0, NWIN, body, 0)

    plsc.subcore_barrier()

    @pl.when(s == 0)
    def _():
        pltpu.sync_copy(deg_sh, zero_v.at[pl.ds(0, N_NODES)])
        pltpu.sync_copy(zero_v, out_hbm.at[pl.ds(c * NP, NP)])


_deg_kernel = functools.partial(
    pl.kernel,
    mesh=_mesh,
    out_type=jax.ShapeDtypeStruct((NC * NP,), jnp.float32),
    scratch_types=[
        pltpu.VMEM((NWIN, WIN), jnp.int32),
        pltpu.VMEM((WIN,), jnp.float32),
        pltpu.VMEM((NP,), jnp.float32),
        pltpu.VMEM_SHARED((N_NODES,), jnp.float32),
        pltpu.SemaphoreType.DMA,
    ],
)(_deg_body)


def _agg_body(hp_hbm, ei2_hbm, ei3_hbm, out_hbm,
              isrc2, idst2, rows0, rows1, wout, acc_sh, sg0, sg1, sz):
    c = lax.axis_index("c")
    s = lax.axis_index("s")
    wid = c * NS + s

    pltpu.async_copy(ei2_hbm.at[0, wid], isrc2, sg0)
    pltpu.async_copy(ei3_hbm.at[1, wid], idst2, sg1)

    for r in range(wout.shape[0]):
        for j in range(CH // 16):
            wout[r, pl.ds(j * 16, 16)] = jnp.zeros((16,), jnp.float32)

    def zb(k, _):
        pltpu.async_copy(wout, acc_sh.at[pl.ds(s * RPT + k * 16, 16)], sz)
        return 0
    lax.fori_loop(0, RPT // 16, zb, 0)
    pltpu.async_copy(wout.at[pl.ds(0, 8)],
                     acc_sh.at[pl.ds(s * RPT + (RPT // 16) * 16, 8)], sz)

    def zdrain(k, _):
        pltpu.make_async_copy(wout, acc_sh.at[pl.ds(s * RPT, 16)], sz).wait()
        return 0
    lax.fori_loop(0, RPT // 16, zdrain, 0)
    pltpu.make_async_copy(wout.at[pl.ds(0, 8)],
                          acc_sh.at[pl.ds(s * RPT, 8)], sz).wait()
    pltpu.make_async_copy(ei2_hbm.at[0, wid], isrc2, sg0).wait()
    pltpu.make_async_copy(ei3_hbm.at[1, wid], idst2, sg1).wait()
    plsc.subcore_barrier()

    def gather(w, buf, sem):
        pltpu.async_copy(hp_hbm.at[isrc2.at[pl.ds(w * WIN, WIN)]], buf, sem)

    def gwait(buf, sem):
        pltpu.make_async_copy(hp_hbm.at[isrc2.at[pl.ds(0, WIN)]], buf,
                              sem).wait()

    def scat(w, buf):
        pltpu.sync_copy(buf, acc_sh.at[idst2.at[w]], add=True)

    gather(0, rows0, sg0)
    gather(1, rows1, sg1)

    def body(k, _):
        w = 2 * k
        gwait(rows0, sg0)
        scat(w, rows0)
        gather(w + 2, rows0, sg0)
        gwait(rows1, sg1)
        scat(w + 1, rows1)
        gather(w + 3, rows1, sg1)
        return 0
    lax.fori_loop(0, (NWIN - 3) // 2, body, 0)

    gwait(rows0, sg0)
    scat(NWIN - 3, rows0)
    gather(NWIN - 1, rows0, sg0)
    gwait(rows1, sg1)
    scat(NWIN - 2, rows1)
    gwait(rows0, sg0)
    scat(NWIN - 1, rows0)

    plsc.subcore_barrier()

    bufs = (rows0, rows1)
    sems = (sg0, sg1)
    sizes = [80] * 7 + [72]
    offs = [80 * k for k in range(8)]
    pltpu.sync_copy(acc_sh.at[pl.ds(s * RPT, 80)], rows0)
    for k in range(8):
        b, sm = bufs[k % 2], sems[k % 2]
        n = sizes[k]
        pltpu.async_copy(b.at[pl.ds(0, n)],
                         out_hbm.at[pl.ds(c * NP + s * RPT + offs[k], n)], sm)
        if k + 1 < 8:
            nb = bufs[(k + 1) % 2]
            pltpu.sync_copy(
                acc_sh.at[pl.ds(s * RPT + offs[k + 1], sizes[k + 1])],
                nb.at[pl.ds(0, sizes[k + 1])])
        pltpu.make_async_copy(
            b.at[pl.ds(0, n)],
            out_hbm.at[pl.ds(c * NP + s * RPT + offs[k], n)], sm).wait()


_agg_kernel = functools.partial(
    pl.kernel,
    mesh=_mesh,
    out_type=jax.ShapeDtypeStruct((NC * NP, CH), jnp.float32),
    scratch_types=[
        pltpu.VMEM((EPW,), jnp.int32),
        pltpu.VMEM((NWIN, WIN), jnp.int32),
        pltpu.VMEM((WIN, CH), jnp.float32),
        pltpu.VMEM((WIN, CH), jnp.float32),
        pltpu.VMEM((16, CH), jnp.float32),
        pltpu.VMEM_SHARED((NP, CH), jnp.float32),
        pltpu.SemaphoreType.DMA,
        pltpu.SemaphoreType.DMA,
        pltpu.SemaphoreType.DMA,
    ],
)(_agg_body)


_R = RPT


def _mm_body(x_ref, w_ref, o_ref):
    o_ref[...] = jnp.dot(x_ref[...], w_ref[...],
                         preferred_element_type=jnp.float32)


def _mm(x, w):
    return pl.pallas_call(
        _mm_body,
        grid=(NP // _R,),
        in_specs=[
            pl.BlockSpec((_R, CH), lambda i: (i, 0)),
            pl.BlockSpec((CH, CH), lambda i: (0, 0)),
        ],
        out_specs=pl.BlockSpec((_R, CH), lambda i: (i, 0)),
        out_shape=jax.ShapeDtypeStruct((NP, CH), jnp.float32),
    )(x, w)


def _scale_body(h_ref, d0_ref, d1_ref, o_ref):
    dis = lax.rsqrt(d0_ref[...] + d1_ref[...] + 1.0)
    o_ref[...] = h_ref[...] * dis


def _scale(h, dg):
    return pl.pallas_call(
        _scale_body,
        grid=(NP // _R,),
        in_specs=[
            pl.BlockSpec((_R, CH), lambda i: (i, 0)),
            pl.BlockSpec((_R, 1), lambda i: (i, 0)),
            pl.BlockSpec((_R, 1), lambda i: (i + NP // _R, 0)),
        ],
        out_specs=pl.BlockSpec((_R, CH), lambda i: (i, 0)),
        out_shape=jax.ShapeDtypeStruct((NP, CH), jnp.float32),
    )(h, dg, dg)


def _comb_mm_body(a0_ref, a1_ref, hp_ref, d0_ref, d1_ref, b_ref, w_ref, o_ref):
    dis = lax.rsqrt(d0_ref[...] + d1_ref[...] + 1.0)
    t = (a0_ref[...] + a1_ref[...] + hp_ref[...]) * dis + b_ref[...]
    t = jnp.maximum(t, 0.0)
    o_ref[...] = jnp.dot(t, w_ref[...],
                         preferred_element_type=jnp.float32) * dis


def _comb_mm(agg, hp, dg, b, w):
    return pl.pallas_call(
        _comb_mm_body,
        grid=(NP // _R,),
        in_specs=[
            pl.BlockSpec((_R, CH), lambda i: (i, 0)),
            pl.BlockSpec((_R, CH), lambda i: (i + NP // _R, 0)),
            pl.BlockSpec((_R, CH), lambda i: (i, 0)),
            pl.BlockSpec((_R, 1), lambda i: (i, 0)),
            pl.BlockSpec((_R, 1), lambda i: (i + NP // _R, 0)),
            pl.BlockSpec((1, CH), lambda i: (0, 0)),
            pl.BlockSpec((CH, CH), lambda i: (0, 0)),
        ],
        out_specs=pl.BlockSpec((_R, CH), lambda i: (i, 0)),
        out_shape=jax.ShapeDtypeStruct((NP, CH), jnp.float32),
    )(agg, agg, hp, dg, dg, b, w)


_OUT_PAD = 8


def _final_body(a0_ref, a1_ref, hp_ref, d0_ref, d1_ref, b_ref, w_ref,
                b3_ref, o_ref):
    dis = lax.rsqrt(d0_ref[...] + d1_ref[...] + 1.0)
    t = (a0_ref[...] + a1_ref[...] + hp_ref[...]) * dis + b_ref[...]
    t = jnp.maximum(t, 0.0)
    o_ref[...] = jnp.dot(t, w_ref[...],
                         preferred_element_type=jnp.float32) + b3_ref[...]


def _final(agg, hp, dg, b, w3p, b3p):
    return pl.pallas_call(
        _final_body,
        grid=(NP // _R,),
        in_specs=[
            pl.BlockSpec((_R, CH), lambda i: (i, 0)),
            pl.BlockSpec((_R, CH), lambda i: (i + NP // _R, 0)),
            pl.BlockSpec((_R, CH), lambda i: (i, 0)),
            pl.BlockSpec((_R, 1), lambda i: (i, 0)),
            pl.BlockSpec((_R, 1), lambda i: (i + NP // _R, 0)),
            pl.BlockSpec((1, CH), lambda i: (0, 0)),
            pl.BlockSpec((CH, _OUT_PAD), lambda i: (0, 0)),
            pl.BlockSpec((1, _OUT_PAD), lambda i: (0, 0)),
        ],
        out_specs=pl.BlockSpec((_R, _OUT_PAD), lambda i: (i, 0)),
        out_shape=jax.ShapeDtypeStruct((NP, _OUT_PAD), jnp.float32),
    )(agg, agg, hp, dg, dg, b, w3p, b3p)


def kernel(x, edge_index, W1, b1, W2, b2, W3, b3):
    ei2 = edge_index.astype(jnp.int32).reshape(2, NW, EPW)
    ei3 = ei2.reshape(2, NW, NWIN, WIN)

    deg = _deg_kernel(ei3)
    dg = deg.reshape(NC * NP, 1)

    b1r = b1.reshape(1, CH)
    b2r = b2.reshape(1, CH)
    w3p = jnp.zeros((CH, _OUT_PAD), jnp.float32).at[:, :3].set(W3)
    b3p = jnp.zeros((1, _OUT_PAD), jnp.float32).at[0, :3].set(b3)

    h1p = _scale(_mm(x, W1), dg)
    agg1 = _agg_kernel(h1p, ei2, ei3)
    h2p = _comb_mm(agg1, h1p, dg, b1r, W2)
    agg2 = _agg_kernel(h2p, ei2, ei3)
    outp = _final(agg2, h2p, dg, b2r, w3p, b3p)
    return outp[:N_NODES, :3]


# speedup vs baseline: 31.6719x; 1.0429x over previous
import functools

import jax
import jax.numpy as jnp
from jax import lax
from jax.experimental import pallas as pl
from jax.experimental.pallas import tpu as pltpu
from jax.experimental.pallas import tpu_sc as plsc

N_NODES = 10000
N_EDGES = 320000
CH = 128
NC = 2
NS = 16
NW = NC * NS
WIN = 128
NWIN = 79
EPW = NWIN * WIN
NP = 10112
RPT = NP // NS
PK_BITS = 14

_mesh = plsc.VectorSubcoreMesh(core_axis_name="c", subcore_axis_name="s")


def _deg_body(pk3_hbm, out_hbm, idx2, dst_u, ones_v, zero_v, deg_sh, sem):
    c = lax.axis_index("c")
    s = lax.axis_index("s")
    wid = c * NS + s

    @pl.when(s == 0)
    def _():
        def zb(i, _):
            zero_v[pl.ds(i * 16, 16)] = jnp.zeros((16,), jnp.float32)
            return 0
        lax.fori_loop(0, NP // 16, zb, 0)
        pltpu.sync_copy(zero_v, deg_sh)

    for j in range(WIN // 16):
        ones_v[pl.ds(j * 16, 16)] = jnp.full((16,), 1.0, jnp.float32)
    pltpu.sync_copy(pk3_hbm.at[wid], idx2)
    plsc.subcore_barrier()

    def body(w, _):
        for j in range(WIN // 16):
            v = idx2[w, pl.ds(j * 16, 16)]
            dst_u[0, pl.ds(j * 16, 16)] = lax.shift_right_logical(v, PK_BITS)
        pltpu.sync_copy(ones_v, deg_sh.at[dst_u.at[0]], add=True)
        return 0
    lax.fori_loop(0, NWIN, body, 0)

    plsc.subcore_barrier()

    @pl.when(s == 0)
    def _():
        pltpu.sync_copy(deg_sh, zero_v)
        pltpu.sync_copy(zero_v, out_hbm.at[pl.ds(c * NP, NP)])


_deg_kernel = functools.partial(
    pl.kernel,
    mesh=_mesh,
    out_type=jax.ShapeDtypeStruct((NC * NP,), jnp.float32),
    scratch_types=[
        pltpu.VMEM((NWIN, WIN), jnp.int32),
        pltpu.VMEM((1, WIN), jnp.int32),
        pltpu.VMEM((WIN,), jnp.float32),
        pltpu.VMEM((NP,), jnp.float32),
        pltpu.VMEM_SHARED((NP,), jnp.float32),
        pltpu.SemaphoreType.DMA,
    ],
)(_deg_body)


def _agg_body(hp_hbm, pk3_hbm, out_hbm,
              ipk2, src_u, dst_u, rows0, rows1, acc_sh, sg0, sg1, sz):
    c = lax.axis_index("c")
    s = lax.axis_index("s")
    wid = c * NS + s

    pltpu.async_copy(pk3_hbm.at[wid], ipk2, sg1)

    def zf(r, _):
        for j in range(CH // 16):
            rows0[r, pl.ds(j * 16, 16)] = jnp.zeros((16,), jnp.float32)
        return 0
    lax.fori_loop(0, WIN, zf, 0)
    for k in range(4):
        pltpu.async_copy(rows0, acc_sh.at[pl.ds(s * RPT + k * 128, 128)], sz)
    pltpu.async_copy(rows0.at[pl.ds(0, 120)],
                     acc_sh.at[pl.ds(s * RPT + 512, 120)], sz)
    for k in range(4):
        pltpu.make_async_copy(rows0, acc_sh.at[pl.ds(s * RPT, 128)], sz).wait()
    pltpu.make_async_copy(rows0.at[pl.ds(0, 120)],
                          acc_sh.at[pl.ds(s * RPT, 120)], sz).wait()
    pltpu.make_async_copy(pk3_hbm.at[wid], ipk2, sg1).wait()
    plsc.subcore_barrier()

    def unpack(w, p):
        for j in range(WIN // 16):
            v = ipk2[w, pl.ds(j * 16, 16)]
            src_u[p, pl.ds(j * 16, 16)] = lax.bitwise_and(
                v, jnp.full((16,), (1 << PK_BITS) - 1, jnp.int32))
            dst_u[p, pl.ds(j * 16, 16)] = lax.shift_right_logical(v, PK_BITS)

    def gather(p, buf, sem):
        pltpu.async_copy(hp_hbm.at[src_u.at[p]], buf, sem)

    def gwait(buf, sem):
        pltpu.make_async_copy(hp_hbm.at[src_u.at[0]], buf, sem).wait()

    def scat(p, buf):
        pltpu.sync_copy(buf, acc_sh.at[dst_u.at[p]], add=True)

    unpack(0, 0)
    gather(0, rows0, sg0)
    unpack(1, 1)
    gather(1, rows1, sg1)

    def body(k, _):
        w = 2 * k
        gwait(rows0, sg0)
        scat(0, rows0)
        unpack(w + 2, 0)
        gather(0, rows0, sg0)
        gwait(rows1, sg1)
        scat(1, rows1)
        unpack(w + 3, 1)
        gather(1, rows1, sg1)
        return 0
    lax.fori_loop(0, (NWIN - 3) // 2, body, 0)

    gwait(rows0, sg0)
    scat(0, rows0)
    unpack(NWIN - 1, 0)
    gather(0, rows0, sg0)
    gwait(rows1, sg1)
    scat(1, rows1)
    gwait(rows0, sg0)
    scat(0, rows0)

    plsc.subcore_barrier()

    bufs = (rows0, rows1)
    sems = (sg0, sg1)
    sizes = [128] * 4 + [120]
    offs = [128 * k for k in range(5)]
    pltpu.sync_copy(acc_sh.at[pl.ds(s * RPT, 128)], rows0)
    for k in range(5):
        b, sm = bufs[k % 2], sems[k % 2]
        n = sizes[k]
        pltpu.async_copy(b.at[pl.ds(0, n)],
                         out_hbm.at[pl.ds(c * NP + s * RPT + offs[k], n)], sm)
        if k + 1 < 5:
            nb = bufs[(k + 1) % 2]
            pltpu.sync_copy(
                acc_sh.at[pl.ds(s * RPT + offs[k + 1], sizes[k + 1])],
                nb.at[pl.ds(0, sizes[k + 1])])
        pltpu.make_async_copy(
            b.at[pl.ds(0, n)],
            out_hbm.at[pl.ds(c * NP + s * RPT + offs[k], n)], sm).wait()


_agg_kernel = functools.partial(
    pl.kernel,
    mesh=_mesh,
    out_type=jax.ShapeDtypeStruct((NC * NP, CH), jnp.float32),
    scratch_types=[
        pltpu.VMEM((NWIN, WIN), jnp.int32),
        pltpu.VMEM((2, WIN), jnp.int32),
        pltpu.VMEM((2, WIN), jnp.int32),
        pltpu.VMEM((WIN, CH), jnp.float32),
        pltpu.VMEM((WIN, CH), jnp.float32),
        pltpu.VMEM_SHARED((NP, CH), jnp.float32),
        pltpu.SemaphoreType.DMA,
        pltpu.SemaphoreType.DMA,
        pltpu.SemaphoreType.DMA,
    ],
)(_agg_body)


_R = RPT


def _mm_body(x_ref, w_ref, o_ref):
    o_ref[...] = jnp.dot(x_ref[...], w_ref[...],
                         preferred_element_type=jnp.float32)


def _mm(x, w):
    return pl.pallas_call(
        _mm_body,
        grid=(NP // _R,),
        in_specs=[
            pl.BlockSpec((_R, CH), lambda i: (i, 0)),
            pl.BlockSpec((CH, CH), lambda i: (0, 0)),
        ],
        out_specs=pl.BlockSpec((_R, CH), lambda i: (i, 0)),
        out_shape=jax.ShapeDtypeStruct((NP, CH), jnp.float32),
    )(x, w)


def _scale_body(h_ref, d0_ref, d1_ref, o_ref):
    dis = lax.rsqrt(d0_ref[...] + d1_ref[...] + 1.0)
    o_ref[...] = h_ref[...] * dis


def _scale(h, dg):
    return pl.pallas_call(
        _scale_body,
        grid=(NP // _R,),
        in_specs=[
            pl.BlockSpec((_R, CH), lambda i: (i, 0)),
            pl.BlockSpec((_R, 1), lambda i: (i, 0)),
            pl.BlockSpec((_R, 1), lambda i: (i + NP // _R, 0)),
        ],
        out_specs=pl.BlockSpec((_R, CH), lambda i: (i, 0)),
        out_shape=jax.ShapeDtypeStruct((NP, CH), jnp.float32),
    )(h, dg, dg)


def _comb_mm_body(a0_ref, a1_ref, hp_ref, d0_ref, d1_ref, b_ref, w_ref, o_ref):
    dis = lax.rsqrt(d0_ref[...] + d1_ref[...] + 1.0)
    t = (a0_ref[...] + a1_ref[...] + hp_ref[...]) * dis + b_ref[...]
    t = jnp.maximum(t, 0.0)
    o_ref[...] = jnp.dot(t, w_ref[...],
                         preferred_element_type=jnp.float32) * dis


def _comb_mm(agg, hp, dg, b, w):
    return pl.pallas_call(
        _comb_mm_body,
        grid=(NP // _R,),
        in_specs=[
            pl.BlockSpec((_R, CH), lambda i: (i, 0)),
            pl.BlockSpec((_R, CH), lambda i: (i + NP // _R, 0)),
            pl.BlockSpec((_R, CH), lambda i: (i, 0)),
            pl.BlockSpec((_R, 1), lambda i: (i, 0)),
            pl.BlockSpec((_R, 1), lambda i: (i + NP // _R, 0)),
            pl.BlockSpec((1, CH), lambda i: (0, 0)),
            pl.BlockSpec((CH, CH), lambda i: (0, 0)),
        ],
        out_specs=pl.BlockSpec((_R, CH), lambda i: (i, 0)),
        out_shape=jax.ShapeDtypeStruct((NP, CH), jnp.float32),
    )(agg, agg, hp, dg, dg, b, w)


_OUT_PAD = 8


def _final_body(a0_ref, a1_ref, hp_ref, d0_ref, d1_ref, b_ref, w_ref,
                b3_ref, o_ref):
    dis = lax.rsqrt(d0_ref[...] + d1_ref[...] + 1.0)
    t = (a0_ref[...] + a1_ref[...] + hp_ref[...]) * dis + b_ref[...]
    t = jnp.maximum(t, 0.0)
    o_ref[...] = jnp.dot(t, w_ref[...],
                         preferred_element_type=jnp.float32) + b3_ref[...]


def _final(agg, hp, dg, b, w3p, b3p):
    return pl.pallas_call(
        _final_body,
        grid=(NP // _R,),
        in_specs=[
            pl.BlockSpec((_R, CH), lambda i: (i, 0)),
            pl.BlockSpec((_R, CH), lambda i: (i + NP // _R, 0)),
            pl.BlockSpec((_R, CH), lambda i: (i, 0)),
            pl.BlockSpec((_R, 1), lambda i: (i, 0)),
            pl.BlockSpec((_R, 1), lambda i: (i + NP // _R, 0)),
            pl.BlockSpec((1, CH), lambda i: (0, 0)),
            pl.BlockSpec((CH, _OUT_PAD), lambda i: (0, 0)),
            pl.BlockSpec((1, _OUT_PAD), lambda i: (0, 0)),
        ],
        out_specs=pl.BlockSpec((_R, _OUT_PAD), lambda i: (i, 0)),
        out_shape=jax.ShapeDtypeStruct((NP, _OUT_PAD), jnp.float32),
    )(agg, agg, hp, dg, dg, b, w3p, b3p)


def kernel(x, edge_index, W1, b1, W2, b2, W3, b3):
    src = edge_index[0].astype(jnp.int32)
    dst = edge_index[1].astype(jnp.int32)
    npad = NW * EPW - N_EDGES
    pidx = jnp.arange(npad, dtype=jnp.int32)
    srcp = jnp.concatenate([src, pidx % 8192])
    dstp = jnp.concatenate([dst, N_NODES + pidx % (NP - N_NODES)])
    packed = ((dstp << PK_BITS) | srcp).reshape(NW, NWIN, WIN)

    deg = _deg_kernel(packed)
    dg = deg.reshape(NC * NP, 1)

    b1r = b1.reshape(1, CH)
    b2r = b2.reshape(1, CH)
    w3p = jnp.zeros((CH, _OUT_PAD), jnp.float32).at[:, :3].set(W3)
    b3p = jnp.zeros((1, _OUT_PAD), jnp.float32).at[0, :3].set(b3)

    h1p = _scale(_mm(x, W1), dg)
    agg1 = _agg_kernel(h1p, packed)
    h2p = _comb_mm(agg1, h1p, dg, b1r, W2)
    agg2 = _agg_kernel(h2p, packed)
    outp = _final(agg2, h2p, dg, b2r, w3p, b3p)
    return outp[:N_NODES, :3]


# speedup vs baseline: 34.3821x; 1.0856x over previous
import functools

import jax
import jax.numpy as jnp
from jax import lax
from jax.experimental import pallas as pl
from jax.experimental.pallas import tpu as pltpu
from jax.experimental.pallas import tpu_sc as plsc

N_NODES = 10000
N_EDGES = 320000
CH = 128
NC = 2
NS = 16
NW = NC * NS
WIN = 128
NWIN = 79
EPW = NWIN * WIN
NP = 10112
RPT = NP // NS
PK_BITS = 14

_mesh = plsc.VectorSubcoreMesh(core_axis_name="c", subcore_axis_name="s")


def _deg_body(pk3_hbm, out_hbm, idx2, dst_u, ones_v, zero_v, deg_sh, sem):
    c = lax.axis_index("c")
    s = lax.axis_index("s")
    wid = c * NS + s

    @pl.when(s == 0)
    def _():
        def zb(i, _):
            zero_v[pl.ds(i * 16, 16)] = jnp.zeros((16,), jnp.float32)
            return 0
        lax.fori_loop(0, NP // 16, zb, 0)
        pltpu.sync_copy(zero_v, deg_sh)

    for j in range(WIN // 16):
        ones_v[pl.ds(j * 16, 16)] = jnp.full((16,), 1.0, jnp.float32)
    pltpu.sync_copy(pk3_hbm.at[wid], idx2)
    plsc.subcore_barrier()

    def body(w, _):
        for j in range(WIN // 16):
            v = idx2[w, pl.ds(j * 16, 16)]
            dst_u[0, pl.ds(j * 16, 16)] = lax.shift_right_logical(v, PK_BITS)
        pltpu.sync_copy(ones_v, deg_sh.at[dst_u.at[0]], add=True)
        return 0
    lax.fori_loop(0, NWIN, body, 0)

    plsc.subcore_barrier()

    @pl.when(s == 0)
    def _():
        pltpu.sync_copy(deg_sh, zero_v)
        pltpu.sync_copy(zero_v, out_hbm.at[pl.ds(c * NP, NP)])


_deg_kernel = functools.partial(
    pl.kernel,
    mesh=_mesh,
    out_type=jax.ShapeDtypeStruct((NC * NP,), jnp.float32),
    scratch_types=[
        pltpu.VMEM((NWIN, WIN), jnp.int32),
        pltpu.VMEM((1, WIN), jnp.int32),
        pltpu.VMEM((WIN,), jnp.float32),
        pltpu.VMEM((NP,), jnp.float32),
        pltpu.VMEM_SHARED((NP,), jnp.float32),
        pltpu.SemaphoreType.DMA,
    ],
)(_deg_body)


def _agg_body(hp_hbm, pk3_hbm, out_hbm,
              ipk2, src_u, dst_u, rows0, rows1, acc_sh, sg0, sg1, sz):
    c = lax.axis_index("c")
    s = lax.axis_index("s")
    wid = c * NS + s

    pltpu.async_copy(pk3_hbm.at[wid], ipk2, sg1)

    def zf(r, _):
        for j in range(CH // 16):
            rows0[r, pl.ds(j * 16, 16)] = jnp.zeros((16,), jnp.float32)
        return 0
    lax.fori_loop(0, WIN, zf, 0)
    for k in range(4):
        pltpu.async_copy(rows0, acc_sh.at[pl.ds(s * RPT + k * 128, 128)], sz)
    pltpu.async_copy(rows0.at[pl.ds(0, 120)],
                     acc_sh.at[pl.ds(s * RPT + 512, 120)], sz)
    for k in range(4):
        pltpu.make_async_copy(rows0, acc_sh.at[pl.ds(s * RPT, 128)], sz).wait()
    pltpu.make_async_copy(rows0.at[pl.ds(0, 120)],
                          acc_sh.at[pl.ds(s * RPT, 120)], sz).wait()
    pltpu.make_async_copy(pk3_hbm.at[wid], ipk2, sg1).wait()
    plsc.subcore_barrier()

    def unpack(w, p):
        for j in range(WIN // 16):
            v = ipk2[w, pl.ds(j * 16, 16)]
            src_u[p, pl.ds(j * 16, 16)] = lax.bitwise_and(
                v, jnp.full((16,), (1 << PK_BITS) - 1, jnp.int32))
            dst_u[p, pl.ds(j * 16, 16)] = lax.shift_right_logical(v, PK_BITS)

    def gather(p, buf, sem):
        pltpu.async_copy(hp_hbm.at[src_u.at[p]], buf, sem)

    def gwait(buf, sem):
        pltpu.make_async_copy(hp_hbm.at[src_u.at[0]], buf, sem).wait()

    def scat(p, buf):
        pltpu.sync_copy(buf, acc_sh.at[dst_u.at[p]], add=True)

    unpack(0, 0)
    gather(0, rows0, sg0)
    unpack(1, 1)
    gather(1, rows1, sg1)

    def body(k, _):
        w = 2 * k
        gwait(rows0, sg0)
        scat(0, rows0)
        unpack(w + 2, 0)
        gather(0, rows0, sg0)
        gwait(rows1, sg1)
        scat(1, rows1)
        unpack(w + 3, 1)
        gather(1, rows1, sg1)
        return 0
    lax.fori_loop(0, (NWIN - 3) // 2, body, 0)

    gwait(rows0, sg0)
    scat(0, rows0)
    unpack(NWIN - 1, 0)
    gather(0, rows0, sg0)
    gwait(rows1, sg1)
    scat(1, rows1)
    gwait(rows0, sg0)
    scat(0, rows0)

    plsc.subcore_barrier()

    bufs = (rows0, rows1)
    sems = (sg0, sg1)
    sizes = [128] * 4 + [120]
    offs = [128 * k for k in range(5)]
    pltpu.sync_copy(acc_sh.at[pl.ds(s * RPT, 128)], rows0)
    for k in range(5):
        b, sm = bufs[k % 2], sems[k % 2]
        n = sizes[k]
        pltpu.async_copy(b.at[pl.ds(0, n)],
                         out_hbm.at[pl.ds(c * NP + s * RPT + offs[k], n)], sm)
        if k + 1 < 5:
            nb = bufs[(k + 1) % 2]
            pltpu.sync_copy(
                acc_sh.at[pl.ds(s * RPT + offs[k + 1], sizes[k + 1])],
                nb.at[pl.ds(0, sizes[k + 1])])
        pltpu.make_async_copy(
            b.at[pl.ds(0, n)],
            out_hbm.at[pl.ds(c * NP + s * RPT + offs[k], n)], sm).wait()


_agg_kernel = functools.partial(
    pl.kernel,
    mesh=_mesh,
    out_type=jax.ShapeDtypeStruct((NC * NP, CH), jnp.float32),
    scratch_types=[
        pltpu.VMEM((NWIN, WIN), jnp.int32),
        pltpu.VMEM((2, WIN), jnp.int32),
        pltpu.VMEM((2, WIN), jnp.int32),
        pltpu.VMEM((WIN, CH), jnp.float32),
        pltpu.VMEM((WIN, CH), jnp.float32),
        pltpu.VMEM_SHARED((NP, CH), jnp.float32),
        pltpu.SemaphoreType.DMA,
        pltpu.SemaphoreType.DMA,
        pltpu.SemaphoreType.DMA,
    ],
)(_agg_body)


_R = 1264
_G = NP // _R


def _dis(d_ref):
    d = lax.rsqrt(d_ref[0, 0:1, :] + d_ref[0, 1:2, :] + 1.0)
    return jnp.swapaxes(d, 0, 1)


def _mm_scale_body(x_ref, w_ref, d_ref, o_ref):
    o_ref[...] = jnp.dot(x_ref[...], w_ref[...],
                         preferred_element_type=jnp.float32) * _dis(d_ref)


def _mm_scale(x, w, dg):
    return pl.pallas_call(
        _mm_scale_body,
        grid=(_G,),
        in_specs=[
            pl.BlockSpec((_R, CH), lambda i: (i, 0)),
            pl.BlockSpec((CH, CH), lambda i: (0, 0)),
            pl.BlockSpec((1, 2, _R), lambda i: (i, 0, 0)),
        ],
        out_specs=pl.BlockSpec((_R, CH), lambda i: (i, 0)),
        out_shape=jax.ShapeDtypeStruct((NP, CH), jnp.float32),
    )(x, w, dg)


def _comb_mm_body(a0_ref, a1_ref, hp_ref, d_ref, b_ref, w_ref, o_ref):
    dis = _dis(d_ref)
    t = (a0_ref[...] + a1_ref[...] + hp_ref[...]) * dis + b_ref[...]
    t = jnp.maximum(t, 0.0)
    o_ref[...] = jnp.dot(t, w_ref[...],
                         preferred_element_type=jnp.float32) * dis


def _comb_mm(agg, hp, dg, b, w):
    return pl.pallas_call(
        _comb_mm_body,
        grid=(_G,),
        in_specs=[
            pl.BlockSpec((_R, CH), lambda i: (i, 0)),
            pl.BlockSpec((_R, CH), lambda i: (i + _G, 0)),
            pl.BlockSpec((_R, CH), lambda i: (i, 0)),
            pl.BlockSpec((1, 2, _R), lambda i: (i, 0, 0)),
            pl.BlockSpec((1, CH), lambda i: (0, 0)),
            pl.BlockSpec((CH, CH), lambda i: (0, 0)),
        ],
        out_specs=pl.BlockSpec((_R, CH), lambda i: (i, 0)),
        out_shape=jax.ShapeDtypeStruct((NP, CH), jnp.float32),
    )(agg, agg, hp, dg, b, w)


_OUT_PAD = 8


def _final_body(a0_ref, a1_ref, hp_ref, d_ref, b_ref, w_ref,
                b3_ref, o_ref):
    dis = _dis(d_ref)
    t = (a0_ref[...] + a1_ref[...] + hp_ref[...]) * dis + b_ref[...]
    t = jnp.maximum(t, 0.0)
    o_ref[...] = jnp.dot(t, w_ref[...],
                         preferred_element_type=jnp.float32) + b3_ref[...]


def _final(agg, hp, dg, b, w3p, b3p):
    return pl.pallas_call(
        _final_body,
        grid=(_G,),
        in_specs=[
            pl.BlockSpec((_R, CH), lambda i: (i, 0)),
            pl.BlockSpec((_R, CH), lambda i: (i + _G, 0)),
            pl.BlockSpec((_R, CH), lambda i: (i, 0)),
            pl.BlockSpec((1, 2, _R), lambda i: (i, 0, 0)),
            pl.BlockSpec((1, CH), lambda i: (0, 0)),
            pl.BlockSpec((CH, _OUT_PAD), lambda i: (0, 0)),
            pl.BlockSpec((1, _OUT_PAD), lambda i: (0, 0)),
        ],
        out_specs=pl.BlockSpec((_R, _OUT_PAD), lambda i: (i, 0)),
        out_shape=jax.ShapeDtypeStruct((NP, _OUT_PAD), jnp.float32),
    )(agg, agg, hp, dg, b, w3p, b3p)


def kernel(x, edge_index, W1, b1, W2, b2, W3, b3):
    src = edge_index[0].astype(jnp.int32)
    dst = edge_index[1].astype(jnp.int32)
    npad = NW * EPW - N_EDGES
    pidx = jnp.arange(npad, dtype=jnp.int32)
    srcp = jnp.concatenate([src, pidx % 8192])
    dstp = jnp.concatenate([dst, N_NODES + pidx % (NP - N_NODES)])
    packed = ((dstp << PK_BITS) | srcp).reshape(NW, NWIN, WIN)

    deg = _deg_kernel(packed)
    dg = deg.reshape(NC, _G, _R).transpose(1, 0, 2)

    b1r = b1.reshape(1, CH)
    b2r = b2.reshape(1, CH)
    w3p = jnp.zeros((CH, _OUT_PAD), jnp.float32).at[:, :3].set(W3)
    b3p = jnp.zeros((1, _OUT_PAD), jnp.float32).at[0, :3].set(b3)

    h1p = _mm_scale(x, W1, dg)
    agg1 = _agg_kernel(h1p, packed)
    h2p = _comb_mm(agg1, h1p, dg, b1r, W2)
    agg2 = _agg_kernel(h2p, packed)
    outp = _final(agg2, h2p, dg, b2r, w3p, b3p)
    return outp[:N_NODES, :3]


# speedup vs baseline: 36.8443x; 1.0716x over previous
import functools

import jax
import jax.numpy as jnp
from jax import lax
from jax.experimental import pallas as pl
from jax.experimental.pallas import tpu as pltpu
from jax.experimental.pallas import tpu_sc as plsc

N_NODES = 10000
N_EDGES = 320000
CH = 128
NC = 2
NS = 16
NW = NC * NS
WIN = 128
NWIN = 79
EPW = NWIN * WIN
NP = 10112
RPT = NP // NS
PK_BITS = 14
NPAD = NW * EPW - N_EDGES

_mesh = plsc.VectorSubcoreMesh(core_axis_name="c", subcore_axis_name="s")


def _deg_body(pk_hbm, out_hbm, ipk, dsts, ones_v, zero_v, deg_sh, sem):
    c = lax.axis_index("c")
    s = lax.axis_index("s")
    wid = c * NS + s

    @pl.when(s == 0)
    def _():
        def zb(i, _):
            zero_v[pl.ds(i * 16, 16)] = jnp.zeros((16,), jnp.float32)
            return 0
        lax.fori_loop(0, NP // 16, zb, 0)
        pltpu.sync_copy(zero_v, deg_sh)

    for j in range(WIN // 16):
        ones_v[pl.ds(j * 16, 16)] = jnp.full((16,), 1.0, jnp.float32)
    pltpu.sync_copy(pk_hbm.at[pl.ds(wid * EPW, EPW)], ipk)

    def ub(w, _):
        for j in range(WIN // 16):
            v = ipk[pl.ds(w * WIN + j * 16, 16)]
            dsts[w, pl.ds(j * 16, 16)] = lax.shift_right_logical(v, PK_BITS)
        return 0
    lax.fori_loop(0, NWIN, ub, 0)
    plsc.subcore_barrier()

    def fire(w, _):
        pltpu.async_copy(ones_v, deg_sh.at[dsts.at[w]], sem, add=True)
        return 0
    lax.fori_loop(0, NWIN, fire, 0)

    def drain(w, _):
        pltpu.make_async_copy(ones_v, deg_sh.at[dsts.at[0]], sem).wait()
        return 0
    lax.fori_loop(0, NWIN, drain, 0)

    plsc.subcore_barrier()

    @pl.when(s == 0)
    def _():
        pltpu.sync_copy(deg_sh, zero_v)
        pltpu.sync_copy(zero_v, out_hbm.at[pl.ds(c * NP, NP)])


_deg_kernel = functools.partial(
    pl.kernel,
    mesh=_mesh,
    out_type=jax.ShapeDtypeStruct((NC * NP,), jnp.float32),
    scratch_types=[
        pltpu.VMEM((EPW,), jnp.int32),
        pltpu.VMEM((NWIN, WIN), jnp.int32),
        pltpu.VMEM((WIN,), jnp.float32),
        pltpu.VMEM((NP,), jnp.float32),
        pltpu.VMEM_SHARED((NP,), jnp.float32),
        pltpu.SemaphoreType.DMA,
    ],
)(_deg_body)


def _agg_body(hp_hbm, pk_hbm, out_hbm,
              ipk2, src_u, dst_u, rows0, rows1, acc_sh, sg0, sg1, sz):
    c = lax.axis_index("c")
    s = lax.axis_index("s")
    wid = c * NS + s

    pltpu.async_copy(pk_hbm.at[pl.ds(wid * EPW, EPW)], ipk2, sg1)

    def zf(r, _):
        for j in range(CH // 16):
            rows0[r, pl.ds(j * 16, 16)] = jnp.zeros((16,), jnp.float32)
        return 0
    lax.fori_loop(0, WIN, zf, 0)
    for k in range(4):
        pltpu.async_copy(rows0, acc_sh.at[pl.ds(s * RPT + k * 128, 128)], sz)
    pltpu.async_copy(rows0.at[pl.ds(0, 120)],
                     acc_sh.at[pl.ds(s * RPT + 512, 120)], sz)
    for k in range(4):
        pltpu.make_async_copy(rows0, acc_sh.at[pl.ds(s * RPT, 128)], sz).wait()
    pltpu.make_async_copy(rows0.at[pl.ds(0, 120)],
                          acc_sh.at[pl.ds(s * RPT, 120)], sz).wait()
    pltpu.make_async_copy(pk_hbm.at[pl.ds(wid * EPW, EPW)], ipk2, sg1).wait()
    plsc.subcore_barrier()

    def unpack(w, p):
        for j in range(WIN // 16):
            v = ipk2[pl.ds(w * WIN + j * 16, 16)]
            src_u[p, pl.ds(j * 16, 16)] = lax.bitwise_and(
                v, jnp.full((16,), (1 << PK_BITS) - 1, jnp.int32))
            dst_u[p, pl.ds(j * 16, 16)] = lax.shift_right_logical(v, PK_BITS)

    def gather(p, buf, sem):
        pltpu.async_copy(hp_hbm.at[src_u.at[p]], buf, sem)

    def gwait(buf, sem):
        pltpu.make_async_copy(hp_hbm.at[src_u.at[0]], buf, sem).wait()

    def scat(p, buf):
        pltpu.sync_copy(buf, acc_sh.at[dst_u.at[p]], add=True)

    unpack(0, 0)
    gather(0, rows0, sg0)
    unpack(1, 1)
    gather(1, rows1, sg1)

    def body(k, _):
        w = 2 * k
        gwait(rows0, sg0)
        scat(0, rows0)
        unpack(w + 2, 0)
        gather(0, rows0, sg0)
        gwait(rows1, sg1)
        scat(1, rows1)
        unpack(w + 3, 1)
        gather(1, rows1, sg1)
        return 0
    lax.fori_loop(0, (NWIN - 3) // 2, body, 0)

    gwait(rows0, sg0)
    scat(0, rows0)
    unpack(NWIN - 1, 0)
    gather(0, rows0, sg0)
    gwait(rows1, sg1)
    scat(1, rows1)
    gwait(rows0, sg0)
    scat(0, rows0)

    plsc.subcore_barrier()

    bufs = (rows0, rows1)
    sems = (sg0, sg1)
    sizes = [128] * 4 + [120]
    offs = [128 * k for k in range(5)]
    pltpu.sync_copy(acc_sh.at[pl.ds(s * RPT, 128)], rows0)
    for k in range(5):
        b, sm = bufs[k % 2], sems[k % 2]
        n = sizes[k]
        pltpu.async_copy(b.at[pl.ds(0, n)],
                         out_hbm.at[pl.ds(c * NP + s * RPT + offs[k], n)], sm)
        if k + 1 < 5:
            nb = bufs[(k + 1) % 2]
            pltpu.sync_copy(
                acc_sh.at[pl.ds(s * RPT + offs[k + 1], sizes[k + 1])],
                nb.at[pl.ds(0, sizes[k + 1])])
        pltpu.make_async_copy(
            b.at[pl.ds(0, n)],
            out_hbm.at[pl.ds(c * NP + s * RPT + offs[k], n)], sm).wait()


_agg_kernel = functools.partial(
    pl.kernel,
    mesh=_mesh,
    out_type=jax.ShapeDtypeStruct((NC * NP, CH), jnp.float32),
    scratch_types=[
        pltpu.VMEM((EPW,), jnp.int32),
        pltpu.VMEM((2, WIN), jnp.int32),
        pltpu.VMEM((2, WIN), jnp.int32),
        pltpu.VMEM((WIN, CH), jnp.float32),
        pltpu.VMEM((WIN, CH), jnp.float32),
        pltpu.VMEM_SHARED((NP, CH), jnp.float32),
        pltpu.SemaphoreType.DMA,
        pltpu.SemaphoreType.DMA,
        pltpu.SemaphoreType.DMA,
    ],
)(_agg_body)


def _pack_body(ei_ref, o_ref):
    sv = ei_ref[0:1, :]
    dv = ei_ref[1:2, :]
    pk = jnp.left_shift(dv, PK_BITS) | sv
    col = lax.broadcasted_iota(jnp.int32, (1, NPAD), 1)
    pads = (jnp.left_shift(N_NODES + col % (NP - N_NODES), PK_BITS)
            | (col % 8192))
    o_ref[...] = jnp.concatenate([pk, pads], axis=1).reshape(NW * EPW)


def _pack(ei):
    return pl.pallas_call(
        _pack_body,
        out_shape=jax.ShapeDtypeStruct((NW * EPW,), jnp.int32),
    )(ei)


_R = 1264
_G = NP // _R


def _dis(d_ref):
    d = lax.rsqrt(d_ref[0, 0:1, :] + d_ref[0, 1:2, :] + 1.0)
    return jnp.swapaxes(d, 0, 1)


def _mm_scale_body(x_ref, w_ref, d_ref, o_ref):
    o_ref[...] = jnp.dot(x_ref[...], w_ref[...],
                         preferred_element_type=jnp.float32) * _dis(d_ref)


def _mm_scale(x, w, dg):
    return pl.pallas_call(
        _mm_scale_body,
        grid=(_G,),
        in_specs=[
            pl.BlockSpec((_R, CH), lambda i: (i, 0)),
            pl.BlockSpec((CH, CH), lambda i: (0, 0)),
            pl.BlockSpec((1, 2, _R), lambda i: (i, 0, 0)),
        ],
        out_specs=pl.BlockSpec((_R, CH), lambda i: (i, 0)),
        out_shape=jax.ShapeDtypeStruct((NP, CH), jnp.float32),
    )(x, w, dg)


def _comb_mm_body(a0_ref, a1_ref, hp_ref, d_ref, b_ref, w_ref, o_ref):
    dis = _dis(d_ref)
    t = (a0_ref[...] + a1_ref[...] + hp_ref[...]) * dis + b_ref[...]
    t = jnp.maximum(t, 0.0)
    o_ref[...] = jnp.dot(t, w_ref[...],
                         preferred_element_type=jnp.float32) * dis


def _comb_mm(agg, hp, dg, b, w):
    return pl.pallas_call(
        _comb_mm_body,
        grid=(_G,),
        in_specs=[
            pl.BlockSpec((_R, CH), lambda i: (i, 0)),
            pl.BlockSpec((_R, CH), lambda i: (i + _G, 0)),
            pl.BlockSpec((_R, CH), lambda i: (i, 0)),
            pl.BlockSpec((1, 2, _R), lambda i: (i, 0, 0)),
            pl.BlockSpec((1, CH), lambda i: (0, 0)),
            pl.BlockSpec((CH, CH), lambda i: (0, 0)),
        ],
        out_specs=pl.BlockSpec((_R, CH), lambda i: (i, 0)),
        out_shape=jax.ShapeDtypeStruct((NP, CH), jnp.float32),
    )(agg, agg, hp, dg, b, w)


_OUT_PAD = 8


def _final_body(a0_ref, a1_ref, hp_ref, d_ref, b_ref, w_ref,
                b3_ref, o_ref):
    dis = _dis(d_ref)
    t = (a0_ref[...] + a1_ref[...] + hp_ref[...]) * dis + b_ref[...]
    t = jnp.maximum(t, 0.0)
    o_ref[...] = jnp.dot(t, w_ref[...],
                         preferred_element_type=jnp.float32) + b3_ref[...]


def _final(agg, hp, dg, b, w3p, b3p):
    return pl.pallas_call(
        _final_body,
        grid=(_G,),
        in_specs=[
            pl.BlockSpec((_R, CH), lambda i: (i, 0)),
            pl.BlockSpec((_R, CH), lambda i: (i + _G, 0)),
            pl.BlockSpec((_R, CH), lambda i: (i, 0)),
            pl.BlockSpec((1, 2, _R), lambda i: (i, 0, 0)),
            pl.BlockSpec((1, CH), lambda i: (0, 0)),
            pl.BlockSpec((CH, _OUT_PAD), lambda i: (0, 0)),
            pl.BlockSpec((1, _OUT_PAD), lambda i: (0, 0)),
        ],
        out_specs=pl.BlockSpec((_R, _OUT_PAD), lambda i: (i, 0)),
        out_shape=jax.ShapeDtypeStruct((NP, _OUT_PAD), jnp.float32),
    )(agg, agg, hp, dg, b, w3p, b3p)


def kernel(x, edge_index, W1, b1, W2, b2, W3, b3):
    packed = _pack(edge_index.astype(jnp.int32))

    deg = _deg_kernel(packed)
    dg = deg.reshape(NC, _G, _R).transpose(1, 0, 2)

    b1r = b1.reshape(1, CH)
    b2r = b2.reshape(1, CH)
    w3p = jnp.zeros((CH, _OUT_PAD), jnp.float32).at[:, :3].set(W3)
    b3p = jnp.zeros((1, _OUT_PAD), jnp.float32).at[0, :3].set(b3)

    h1p = _mm_scale(x, W1, dg)
    agg1 = _agg_kernel(h1p, packed)
    h2p = _comb_mm(agg1, h1p, dg, b1r, W2)
    agg2 = _agg_kernel(h2p, packed)
    outp = _final(agg2, h2p, dg, b2r, w3p, b3p)
    return outp[:N_NODES, :3]
